# baseline (device time: 76062 ns/iter reference)
import jax
import jax.numpy as jnp
from jax import lax
from jax.experimental import pallas as pl
from jax.experimental.pallas import tpu as pltpu

N_DEV = 16
B, SQ, D = 4, 256, 1024
H_LOC, DH = 8, 128
ROWS = B * SQ
SCALE = 0.08838834764831843

MASKS = (
    (1, 3, 4, 8),
    (4, 8, 1, 3),
)
CW = D // 2
HR = ROWS // 2
LENS = (256, 128, 64, 32, 32, 64, 128, 256)
OFFS = (0, 256, 384, 448)
BUF_ROWS = 480


def _mask_of(st, s):
    return MASKS[st][s] if s < 4 else MASKS[st][7 - s]


def kernel(x, Wq, Wo, Wk, Wv):
    x2 = x.reshape(ROWS, D)

    def body(x_ref, wq_ref, wk_ref, wv_ref, wo_ref, out_ref,
             q_ref, k_ref, v_ref, o_ref, acc_ref,
             wqb_ref, wkb_ref, wvb_ref, wob_ref,
             recv_buf, send_sems, recv_sems):
        my = lax.axis_index("i")
        i0 = my & 1
        i1 = (my >> 1) & 1
        i2 = (my >> 2) & 1
        i3 = (my >> 3) & 1
        bits = {1: i0 ^ i1, 3: i1, 4: i2, 8: i3}
        partner = {m: my ^ m for m in (1, 3, 4, 8)}

        wqb_ref[...] = wq_ref[...].astype(jnp.bfloat16)
        wkb_ref[...] = wk_ref[...].astype(jnp.bfloat16)
        wvb_ref[...] = wv_ref[...].astype(jnp.bfloat16)
        wob_ref[...] = wo_ref[...].astype(jnp.bfloat16)

        def compute_half(h):
            hr = slice(h * HR, (h + 1) * HR)
            xb = x_ref[hr, :].astype(jnp.bfloat16)
            q_ref[hr, :] = jnp.dot(xb, wqb_ref[...],
                                   preferred_element_type=jnp.float32
                                   ).astype(jnp.bfloat16)
            k_ref[hr, :] = jnp.dot(xb, wkb_ref[...],
                                   preferred_element_type=jnp.float32
                                   ).astype(jnp.bfloat16)
            v_ref[hr, :] = jnp.dot(xb, wvb_ref[...],
                                   preferred_element_type=jnp.float32
                                   ).astype(jnp.bfloat16)
            for b in (2 * h, 2 * h + 1):
                r0 = b * SQ
                for hd in range(H_LOC):
                    c0 = hd * DH
                    q = q_ref[r0:r0 + SQ, c0:c0 + DH]
                    k = k_ref[r0:r0 + SQ, c0:c0 + DH]
                    v = v_ref[r0:r0 + SQ, c0:c0 + DH]
                    s = lax.dot_general(
                        q, k, (((1,), (1,)), ((), ())),
                        preferred_element_type=jnp.float32) * SCALE
                    m = jnp.max(s, axis=1, keepdims=True)
                    p = jnp.exp(s - m)
                    l = jnp.sum(p, axis=1, keepdims=True)
                    o_ref[r0:r0 + SQ, c0:c0 + DH] = jnp.dot(
                        (p / l).astype(jnp.bfloat16), v,
                        preferred_element_type=jnp.float32
                    ).astype(jnp.bfloat16)
            acc_ref[hr, :] = jnp.dot(
                o_ref[hr, :], wob_ref[...],
                preferred_element_type=jnp.float32).astype(jnp.bfloat16)

        lo = [[jnp.int32(0), jnp.int32(0)], [jnp.int32(HR), jnp.int32(HR)]]
        rdmas = {}

        def begin(bf, st, s):
            m = _mask_of(st, s)
            L = LENS[s]
            c0 = st * CW
            idx = bf * 2 + st
            if s < 4:
                src_lo = lo[bf][st] + (1 - bits[m]) * L
                src_ref = acc_ref.at[pl.ds(src_lo, L), pl.ds(c0, CW)]
                dst_ref = recv_buf.at[idx, pl.ds(OFFS[s], L), :]
            else:
                src_ref = out_ref.at[pl.ds(lo[bf][st], L), pl.ds(c0, CW)]
                dst_ref = out_ref.at[pl.ds(lo[bf][st], L), pl.ds(c0, CW)]
            rdma = pltpu.make_async_remote_copy(
                src_ref=src_ref,
                dst_ref=dst_ref,
                send_sem=send_sems.at[idx * 8 + s],
                recv_sem=recv_sems.at[idx * 8 + s],
                device_id=(partner[m],),
                device_id_type=pl.DeviceIdType.MESH,
            )
            rdma.start()
            rdmas[(bf, st, s)] = rdma

        def finish(bf, st, s):
            m = _mask_of(st, s)
            L = LENS[s]
            c0 = st * CW
            b = bits[m]
            idx = bf * 2 + st
            rdmas[(bf, st, s)].wait()
            if s < 4:
                recv = recv_buf[idx, OFFS[s]:OFFS[s] + L, :]
                keep_lo = lo[bf][st] + b * L
                acc_ref[pl.ds(keep_lo, L), c0:c0 + CW] = (
                    acc_ref[pl.ds(keep_lo, L), c0:c0 + CW].astype(jnp.float32)
                    + recv.astype(jnp.float32)).astype(jnp.bfloat16)
                lo[bf][st] = keep_lo
                if s == 3:
                    out_ref[pl.ds(keep_lo, 32), c0:c0 + CW] = (
                        acc_ref[pl.ds(keep_lo, 32), c0:c0 + CW].astype(
                            jnp.float32))
            else:
                lo[bf][st] = lo[bf][st] - b * L

        compute_half(0)

        barrier = pltpu.get_barrier_semaphore()
        for m in (1, 3, 4, 8):
            pl.semaphore_signal(barrier, inc=1, device_id=(partner[m],),
                                device_id_type=pl.DeviceIdType.MESH)
        pl.semaphore_wait(barrier, 4)

        for st in range(2):
            begin(0, st, 0)
        compute_half(1)
        for st in range(2):
            begin(1, st, 0)
        for s in range(8):
            for st in range(2):
                finish(0, st, s)
                if s < 7:
                    begin(0, st, s + 1)
            for st in range(2):
                finish(1, st, s)
                if s < 7:
                    begin(1, st, s + 1)

    out = pl.pallas_call(
        body,
        out_shape=jax.ShapeDtypeStruct((ROWS, D), jnp.float32),
        in_specs=[pl.BlockSpec(memory_space=pltpu.VMEM)] * 5,
        out_specs=pl.BlockSpec(memory_space=pltpu.VMEM),
        scratch_shapes=[
            pltpu.VMEM((ROWS, D), jnp.bfloat16),
            pltpu.VMEM((ROWS, D), jnp.bfloat16),
            pltpu.VMEM((ROWS, D), jnp.bfloat16),
            pltpu.VMEM((ROWS, D), jnp.bfloat16),
            pltpu.VMEM((ROWS, D), jnp.bfloat16),
            pltpu.VMEM((D, D), jnp.bfloat16),
            pltpu.VMEM((D, D), jnp.bfloat16),
            pltpu.VMEM((D, D), jnp.bfloat16),
            pltpu.VMEM((D, D), jnp.bfloat16),
            pltpu.VMEM((4, BUF_ROWS, CW), jnp.bfloat16),
            pltpu.SemaphoreType.DMA((32,)),
            pltpu.SemaphoreType.DMA((32,)),
        ],
        compiler_params=pltpu.CompilerParams(
            collective_id=0, vmem_limit_bytes=100 * 1024 * 1024),
    )(x2, Wq, Wk, Wv, Wo)
    return out.reshape(B, SQ, D)


# device time: 63770 ns/iter; 1.1928x vs baseline; 1.1928x over previous
import jax
import jax.numpy as jnp
from jax import lax
from jax.experimental import pallas as pl
from jax.experimental.pallas import tpu as pltpu

N_DEV = 16
B, SQ, D = 4, 256, 1024
H_LOC, DH = 8, 128
ROWS = B * SQ
SCALE = 0.08838834764831843

MASKS = (
    (1, 3, 4, 8),
    (4, 8, 1, 3),
)
CW = D // 2
HR = ROWS // 2
LENS = (256, 128, 64, 32, 32, 64, 128, 256)
OFFS = (0, 256, 384, 448)
BUF_ROWS = 480


def _mask_of(st, s):
    return MASKS[st][s] if s < 4 else MASKS[st][7 - s]


def kernel(x, Wq, Wo, Wk, Wv):
    x2 = x.reshape(ROWS, D)

    def body(x_ref, wq_ref, wk_ref, wv_ref, wo_ref, out_ref,
             q_ref, k_ref, v_ref, o_ref,
             wqb_ref, wkb_ref, wvb_ref, wob_ref,
             acc_ws, ag_ws, recv_buf, send_sems, recv_sems):
        my = lax.axis_index("i")
        i0 = my & 1
        i1 = (my >> 1) & 1
        i2 = (my >> 2) & 1
        i3 = (my >> 3) & 1
        bits = {1: i0 ^ i1, 3: i1, 4: i2, 8: i3}
        partner = {m: my ^ m for m in (1, 3, 4, 8)}

        barrier = pltpu.get_barrier_semaphore()
        for m in (1, 3, 4, 8):
            pl.semaphore_signal(barrier, inc=1, device_id=(partner[m],),
                                device_id_type=pl.DeviceIdType.MESH)
        pl.semaphore_wait(barrier, 4)

        wqb_ref[...] = wq_ref[...].astype(jnp.bfloat16)
        wkb_ref[...] = wk_ref[...].astype(jnp.bfloat16)
        wvb_ref[...] = wv_ref[...].astype(jnp.bfloat16)
        wob_ref[...] = wo_ref[...].astype(jnp.bfloat16)

        def compute_half(h):
            hr = slice(h * HR, (h + 1) * HR)
            xb = x_ref[hr, :].astype(jnp.bfloat16)
            q_ref[hr, :] = jnp.dot(xb, wqb_ref[...],
                                   preferred_element_type=jnp.float32
                                   ).astype(jnp.bfloat16)
            k_ref[hr, :] = jnp.dot(xb, wkb_ref[...],
                                   preferred_element_type=jnp.float32
                                   ).astype(jnp.bfloat16)
            v_ref[hr, :] = jnp.dot(xb, wvb_ref[...],
                                   preferred_element_type=jnp.float32
                                   ).astype(jnp.bfloat16)
            for b in (2 * h, 2 * h + 1):
                r0 = b * SQ
                for hd in range(H_LOC):
                    c0 = hd * DH
                    q = q_ref[r0:r0 + SQ, c0:c0 + DH]
                    k = k_ref[r0:r0 + SQ, c0:c0 + DH]
                    v = v_ref[r0:r0 + SQ, c0:c0 + DH]
                    s = lax.dot_general(
                        q, k, (((1,), (1,)), ((), ())),
                        preferred_element_type=jnp.float32) * SCALE
                    m = jnp.max(s, axis=1, keepdims=True)
                    p = jnp.exp(s - m)
                    l = jnp.sum(p, axis=1, keepdims=True)
                    o_ref[r0:r0 + SQ, c0:c0 + DH] = jnp.dot(
                        (p / l).astype(jnp.bfloat16), v,
                        preferred_element_type=jnp.float32
                    ).astype(jnp.bfloat16)
            for st in range(2):
                acc_ws[h * 2 + st, :, :] = jnp.dot(
                    o_ref[hr, :], wob_ref[:, st * CW:(st + 1) * CW],
                    preferred_element_type=jnp.float32).astype(jnp.bfloat16)

        lo = [[jnp.int32(0), jnp.int32(0)], [jnp.int32(0), jnp.int32(0)]]
        rdmas = {}
        pending = []

        def begin(bf, st, s):
            m = _mask_of(st, s)
            L = LENS[s]
            idx = bf * 2 + st
            if s < 4:
                src_lo = lo[bf][st] + (1 - bits[m]) * L
                src_ref = acc_ws.at[idx, pl.ds(src_lo, L), :]
                dst_ref = recv_buf.at[idx, pl.ds(OFFS[s], L), :]
            else:
                src_ref = ag_ws.at[idx, pl.ds(lo[bf][st], L), :]
                dst_ref = ag_ws.at[idx, pl.ds(lo[bf][st], L), :]
            rdma = pltpu.make_async_remote_copy(
                src_ref=src_ref,
                dst_ref=dst_ref,
                send_sem=send_sems.at[idx * 8 + s],
                recv_sem=recv_sems.at[idx * 8 + s],
                device_id=(partner[m],),
                device_id_type=pl.DeviceIdType.MESH,
            )
            rdma.start()
            rdmas[(bf, st, s)] = rdma

        def finish(bf, st, s):
            m = _mask_of(st, s)
            L = LENS[s]
            b = bits[m]
            idx = bf * 2 + st
            rdmas[(bf, st, s)].wait()
            if s < 4:
                keep_lo = lo[bf][st] + b * L
                acc_ws[idx, pl.ds(keep_lo, L), :] = (
                    acc_ws[idx, pl.ds(keep_lo, L), :].astype(jnp.float32)
                    + recv_buf[idx, OFFS[s]:OFFS[s] + L, :].astype(
                        jnp.float32)).astype(jnp.bfloat16)
                lo[bf][st] = keep_lo
                if s == 3:
                    ag_ws[idx, pl.ds(keep_lo, 32), :] = (
                        acc_ws[idx, pl.ds(keep_lo, 32), :])
                    pending.append((bf, st, keep_lo, 32))
            else:
                recv_lo = lo[bf][st] + (1 - 2 * b) * L
                lo[bf][st] = lo[bf][st] - b * L
                pending.append((bf, st, recv_lo, L))

        def flush_pending():
            for bf, st, w_lo, L in pending:
                idx = bf * 2 + st
                out_ref[pl.ds(bf * HR + w_lo, L), st * CW:(st + 1) * CW] = (
                    ag_ws[idx, pl.ds(w_lo, L), :].astype(jnp.float32))
            pending.clear()

        compute_half(0)
        for st in range(2):
            begin(0, st, 0)
        compute_half(1)
        for st in range(2):
            begin(1, st, 0)
        for s in range(8):
            for st in range(2):
                finish(0, st, s)
                if s < 7:
                    begin(0, st, s + 1)
            for st in range(2):
                finish(1, st, s)
                if s < 7:
                    begin(1, st, s + 1)
            flush_pending()

    out = pl.pallas_call(
        body,
        out_shape=jax.ShapeDtypeStruct((ROWS, D), jnp.float32),
        in_specs=[pl.BlockSpec(memory_space=pltpu.VMEM)] * 5,
        out_specs=pl.BlockSpec(memory_space=pltpu.VMEM),
        scratch_shapes=[
            pltpu.VMEM((ROWS, D), jnp.bfloat16),
            pltpu.VMEM((ROWS, D), jnp.bfloat16),
            pltpu.VMEM((ROWS, D), jnp.bfloat16),
            pltpu.VMEM((ROWS, D), jnp.bfloat16),
            pltpu.VMEM((D, D), jnp.bfloat16),
            pltpu.VMEM((D, D), jnp.bfloat16),
            pltpu.VMEM((D, D), jnp.bfloat16),
            pltpu.VMEM((D, D), jnp.bfloat16),
            pltpu.VMEM((4, HR, CW), jnp.bfloat16),
            pltpu.VMEM((4, HR, CW), jnp.bfloat16),
            pltpu.VMEM((4, BUF_ROWS, CW), jnp.bfloat16),
            pltpu.SemaphoreType.DMA((32,)),
            pltpu.SemaphoreType.DMA((32,)),
        ],
        compiler_params=pltpu.CompilerParams(
            collective_id=0, vmem_limit_bytes=100 * 1024 * 1024),
    )(x2, Wq, Wk, Wv, Wo)
    return out.reshape(B, SQ, D)


# device time: 63656 ns/iter; 1.1949x vs baseline; 1.0018x over previous
import jax
import jax.numpy as jnp
from jax import lax
from jax.experimental import pallas as pl
from jax.experimental.pallas import tpu as pltpu

N_DEV = 16
B, SQ, D = 4, 256, 1024
H_LOC, DH = 8, 128
ROWS = B * SQ
SCALE = 0.08838834764831843

MASKS = (
    (1, 3, 4, 8),
    (4, 8, 1, 3),
)
CW = D // 2
HR = ROWS // 2
LENS = (256, 128, 64, 32, 32, 64, 128, 256)
OFFS = (0, 256, 384, 448)
BUF_ROWS = 480


def _mask_of(st, s):
    return MASKS[st][s] if s < 4 else MASKS[st][7 - s]


def kernel(x, Wq, Wo, Wk, Wv):
    x2 = x.reshape(ROWS, D)

    def body(x_ref, wq_ref, wk_ref, wv_ref, wo_ref, out_ref,
             q_ref, k_ref, v_ref, o_ref,
             wqb_ref, wkb_ref, wvb_ref, wob_ref,
             acc_ws, ag_ws, recv_buf, send_sems, recv_sems):
        my = lax.axis_index("i")
        i0 = my & 1
        i1 = (my >> 1) & 1
        i2 = (my >> 2) & 1
        i3 = (my >> 3) & 1
        bits = {1: i0 ^ i1, 3: i1, 4: i2, 8: i3}
        partner = {m: my ^ m for m in (1, 3, 4, 8)}

        barrier = pltpu.get_barrier_semaphore()
        for m in (1, 3, 4, 8):
            pl.semaphore_signal(barrier, inc=1, device_id=(partner[m],),
                                device_id_type=pl.DeviceIdType.MESH)
        pl.semaphore_wait(barrier, 4)

        wqb_ref[...] = wq_ref[...].astype(jnp.bfloat16)
        wkb_ref[...] = wk_ref[...].astype(jnp.bfloat16)
        wvb_ref[...] = wv_ref[...].astype(jnp.bfloat16)
        wob_ref[...] = wo_ref[...].astype(jnp.bfloat16)

        def compute_half(h):
            hr = slice(h * HR, (h + 1) * HR)
            xb = x_ref[hr, :].astype(jnp.bfloat16)
            q_ref[hr, :] = jnp.dot(xb, wqb_ref[...],
                                   preferred_element_type=jnp.float32
                                   ).astype(jnp.bfloat16)
            k_ref[hr, :] = jnp.dot(xb, wkb_ref[...],
                                   preferred_element_type=jnp.float32
                                   ).astype(jnp.bfloat16)
            v_ref[hr, :] = jnp.dot(xb, wvb_ref[...],
                                   preferred_element_type=jnp.float32
                                   ).astype(jnp.bfloat16)
            for b in (2 * h, 2 * h + 1):
                r0 = b * SQ
                for hd in range(H_LOC):
                    c0 = hd * DH
                    q = q_ref[r0:r0 + SQ, c0:c0 + DH]
                    k = k_ref[r0:r0 + SQ, c0:c0 + DH]
                    v = v_ref[r0:r0 + SQ, c0:c0 + DH]
                    s = lax.dot_general(
                        q, k, (((1,), (1,)), ((), ())),
                        preferred_element_type=jnp.float32) * SCALE
                    m = jnp.max(s, axis=1, keepdims=True)
                    p = jnp.exp(s - m)
                    l = jnp.sum(p, axis=1, keepdims=True)
                    o_ref[r0:r0 + SQ, c0:c0 + DH] = jnp.dot(
                        (p / l).astype(jnp.bfloat16), v,
                        preferred_element_type=jnp.float32
                    ).astype(jnp.bfloat16)
            for st in range(2):
                acc_ws[h * 2 + st, :, :] = jnp.dot(
                    o_ref[hr, :], wob_ref[:, st * CW:(st + 1) * CW],
                    preferred_element_type=jnp.float32).astype(jnp.bfloat16)

        lo = [[jnp.int32(0), jnp.int32(0)], [jnp.int32(0), jnp.int32(0)]]
        rdmas = {}
        pending = []

        def begin(bf, st, s):
            m = _mask_of(st, s)
            L = LENS[s]
            idx = bf * 2 + st
            if s < 4:
                src_lo = lo[bf][st] + (1 - bits[m]) * L
                src_ref = acc_ws.at[idx, pl.ds(src_lo, L), :]
                dst_ref = recv_buf.at[idx, pl.ds(OFFS[s], L), :]
            else:
                src_ref = ag_ws.at[idx, pl.ds(lo[bf][st], L), :]
                dst_ref = ag_ws.at[idx, pl.ds(lo[bf][st], L), :]
            rdma = pltpu.make_async_remote_copy(
                src_ref=src_ref,
                dst_ref=dst_ref,
                send_sem=send_sems.at[idx * 8 + s],
                recv_sem=recv_sems.at[idx * 8 + s],
                device_id=(partner[m],),
                device_id_type=pl.DeviceIdType.MESH,
            )
            rdma.start()
            rdmas[(bf, st, s)] = rdma

        def finish(bf, st, s):
            m = _mask_of(st, s)
            L = LENS[s]
            b = bits[m]
            idx = bf * 2 + st
            rdmas[(bf, st, s)].wait()
            if s < 4:
                keep_lo = lo[bf][st] + b * L
                acc_ws[idx, pl.ds(keep_lo, L), :] = (
                    acc_ws[idx, pl.ds(keep_lo, L), :].astype(jnp.float32)
                    + recv_buf[idx, OFFS[s]:OFFS[s] + L, :].astype(
                        jnp.float32)).astype(jnp.bfloat16)
                lo[bf][st] = keep_lo
                if s == 3:
                    ag_ws[idx, pl.ds(keep_lo, 32), :] = (
                        acc_ws[idx, pl.ds(keep_lo, 32), :])
                    pending.append((bf, st, keep_lo, 32))
            else:
                recv_lo = lo[bf][st] + (1 - 2 * b) * L
                lo[bf][st] = lo[bf][st] - b * L
                pending.append((bf, st, recv_lo, L))

        def flush_pending():
            for bf, st, w_lo, L in pending:
                idx = bf * 2 + st
                out_ref[pl.ds(bf * HR + w_lo, L), st * CW:(st + 1) * CW] = (
                    ag_ws[idx, pl.ds(w_lo, L), :].astype(jnp.float32))
            pending.clear()

        compute_half(0)
        for st in range(2):
            begin(0, st, 0)
        compute_half(1)
        for st in range(2):
            begin(1, st, 0)
        for st in range(2):
            finish(0, st, 0)
            begin(0, st, 1)
        for r in range(8):
            for st in range(2):
                finish(1, st, r)
                if r < 7:
                    begin(1, st, r + 1)
            if r + 1 <= 7:
                for st in range(2):
                    finish(0, st, r + 1)
                    if r + 2 <= 7:
                        begin(0, st, r + 2)
            flush_pending()

    out = pl.pallas_call(
        body,
        out_shape=jax.ShapeDtypeStruct((ROWS, D), jnp.float32),
        in_specs=[pl.BlockSpec(memory_space=pltpu.VMEM)] * 5,
        out_specs=pl.BlockSpec(memory_space=pltpu.VMEM),
        scratch_shapes=[
            pltpu.VMEM((ROWS, D), jnp.bfloat16),
            pltpu.VMEM((ROWS, D), jnp.bfloat16),
            pltpu.VMEM((ROWS, D), jnp.bfloat16),
            pltpu.VMEM((ROWS, D), jnp.bfloat16),
            pltpu.VMEM((D, D), jnp.bfloat16),
            pltpu.VMEM((D, D), jnp.bfloat16),
            pltpu.VMEM((D, D), jnp.bfloat16),
            pltpu.VMEM((D, D), jnp.bfloat16),
            pltpu.VMEM((4, HR, CW), jnp.bfloat16),
            pltpu.VMEM((4, HR, CW), jnp.bfloat16),
            pltpu.VMEM((4, BUF_ROWS, CW), jnp.bfloat16),
            pltpu.SemaphoreType.DMA((32,)),
            pltpu.SemaphoreType.DMA((32,)),
        ],
        compiler_params=pltpu.CompilerParams(
            collective_id=0, vmem_limit_bytes=100 * 1024 * 1024),
    )(x2, Wq, Wk, Wv, Wo)
    return out.reshape(B, SQ, D)


# device time: 58541 ns/iter; 1.2993x vs baseline; 1.0874x over previous
import jax
import jax.numpy as jnp
from jax import lax
from jax.experimental import pallas as pl
from jax.experimental.pallas import tpu as pltpu

N_DEV = 16
B, SQ, D = 4, 256, 1024
H_LOC, DH = 8, 128
ROWS = B * SQ
SCALE = 0.08838834764831843

MASKS = (
    (1, 3, 4, 8),
    (4, 8, 1, 3),
)
CW = D // 2
BR = SQ
NBF = 4
LENS = (128, 64, 32, 16, 16, 32, 64, 128)
OFFS = (0, 128, 192, 224)
BUF_ROWS = 240


def _mask_of(st, s):
    return MASKS[st][s] if s < 4 else MASKS[st][7 - s]


def kernel(x, Wq, Wo, Wk, Wv):
    x2 = x.reshape(ROWS, D)

    def body(x_ref, wq_ref, wk_ref, wv_ref, wo_ref, out_ref,
             q_ref, k_ref, v_ref, o_ref,
             wqb_ref, wkb_ref, wvb_ref, wob_ref,
             acc_ws, ag_ws, recv_buf, send_sems, recv_sems):
        my = lax.axis_index("i")
        i0 = my & 1
        i1 = (my >> 1) & 1
        i2 = (my >> 2) & 1
        i3 = (my >> 3) & 1
        bits = {1: i0 ^ i1, 3: i1, 4: i2, 8: i3}
        partner = {m: my ^ m for m in (1, 3, 4, 8)}

        barrier = pltpu.get_barrier_semaphore()
        for m in (1, 3, 4, 8):
            pl.semaphore_signal(barrier, inc=1, device_id=(partner[m],),
                                device_id_type=pl.DeviceIdType.MESH)
        pl.semaphore_wait(barrier, 4)

        wqb_ref[...] = wq_ref[...].astype(jnp.bfloat16)
        wkb_ref[...] = wk_ref[...].astype(jnp.bfloat16)
        wvb_ref[...] = wv_ref[...].astype(jnp.bfloat16)
        wob_ref[...] = wo_ref[...].astype(jnp.bfloat16)

        def compute_batch(b):
            hr = slice(b * BR, (b + 1) * BR)
            xb = x_ref[hr, :].astype(jnp.bfloat16)
            q_ref[hr, :] = jnp.dot(xb, wqb_ref[...],
                                   preferred_element_type=jnp.float32
                                   ).astype(jnp.bfloat16)
            k_ref[hr, :] = jnp.dot(xb, wkb_ref[...],
                                   preferred_element_type=jnp.float32
                                   ).astype(jnp.bfloat16)
            v_ref[hr, :] = jnp.dot(xb, wvb_ref[...],
                                   preferred_element_type=jnp.float32
                                   ).astype(jnp.bfloat16)
            r0 = b * SQ
            for hd in range(H_LOC):
                c0 = hd * DH
                q = q_ref[r0:r0 + SQ, c0:c0 + DH]
                k = k_ref[r0:r0 + SQ, c0:c0 + DH]
                v = v_ref[r0:r0 + SQ, c0:c0 + DH]
                s = lax.dot_general(
                    q, k, (((1,), (1,)), ((), ())),
                    preferred_element_type=jnp.float32) * SCALE
                m = jnp.max(s, axis=1, keepdims=True)
                p = jnp.exp(s - m)
                l = jnp.sum(p, axis=1, keepdims=True)
                o_ref[r0:r0 + SQ, c0:c0 + DH] = jnp.dot(
                    (p / l).astype(jnp.bfloat16), v,
                    preferred_element_type=jnp.float32
                ).astype(jnp.bfloat16)
            for st in range(2):
                acc_ws[b * 2 + st, :, :] = jnp.dot(
                    o_ref[hr, :], wob_ref[:, st * CW:(st + 1) * CW],
                    preferred_element_type=jnp.float32).astype(jnp.bfloat16)

        lo = [[jnp.int32(0), jnp.int32(0)] for _ in range(NBF)]
        rdmas = {}
        pending = []

        def begin(bf, st, s):
            m = _mask_of(st, s)
            L = LENS[s]
            idx = bf * 2 + st
            if s < 4:
                src_lo = lo[bf][st] + (1 - bits[m]) * L
                src_ref = acc_ws.at[idx, pl.ds(src_lo, L), :]
                dst_ref = recv_buf.at[idx, pl.ds(OFFS[s], L), :]
            else:
                src_ref = ag_ws.at[idx, pl.ds(lo[bf][st], L), :]
                dst_ref = ag_ws.at[idx, pl.ds(lo[bf][st], L), :]
            rdma = pltpu.make_async_remote_copy(
                src_ref=src_ref,
                dst_ref=dst_ref,
                send_sem=send_sems.at[idx * 8 + s],
                recv_sem=recv_sems.at[idx * 8 + s],
                device_id=(partner[m],),
                device_id_type=pl.DeviceIdType.MESH,
            )
            rdma.start()
            rdmas[(bf, st, s)] = rdma

        def finish(bf, st, s):
            m = _mask_of(st, s)
            L = LENS[s]
            b = bits[m]
            idx = bf * 2 + st
            rdmas[(bf, st, s)].wait()
            if s < 4:
                keep_lo = lo[bf][st] + b * L
                acc_ws[idx, pl.ds(keep_lo, L), :] = (
                    acc_ws[idx, pl.ds(keep_lo, L), :].astype(jnp.float32)
                    + recv_buf[idx, OFFS[s]:OFFS[s] + L, :].astype(
                        jnp.float32)).astype(jnp.bfloat16)
                lo[bf][st] = keep_lo
                if s == 3:
                    ag_ws[idx, pl.ds(keep_lo, 16), :] = (
                        acc_ws[idx, pl.ds(keep_lo, 16), :])
                    pending.append((bf, st, keep_lo, 16))
            else:
                recv_lo = lo[bf][st] + (1 - 2 * b) * L
                lo[bf][st] = lo[bf][st] - b * L
                pending.append((bf, st, recv_lo, L))

        def step(bf, st, s):
            finish(bf, st, s)
            if s < 7:
                begin(bf, st, s + 1)

        def flush_pending():
            for bf, st, w_lo, L in pending:
                idx = bf * 2 + st
                out_ref[pl.ds(bf * BR + w_lo, L), st * CW:(st + 1) * CW] = (
                    ag_ws[idx, pl.ds(w_lo, L), :].astype(jnp.float32))
            pending.clear()

        for b in range(B):
            compute_batch(b)
            for st in range(2):
                begin(b, st, 0)
            for older in range(b):
                for st in range(2):
                    step(older, st, b - 1 - older)
        for r in range(8):
            for bf in range(NBF - 1, -1, -1):
                s = r + (NBF - 1 - bf)
                if s <= 7:
                    for st in range(2):
                        step(bf, st, s)
            flush_pending()

    out = pl.pallas_call(
        body,
        out_shape=jax.ShapeDtypeStruct((ROWS, D), jnp.float32),
        in_specs=[pl.BlockSpec(memory_space=pltpu.VMEM)] * 5,
        out_specs=pl.BlockSpec(memory_space=pltpu.VMEM),
        scratch_shapes=[
            pltpu.VMEM((ROWS, D), jnp.bfloat16),
            pltpu.VMEM((ROWS, D), jnp.bfloat16),
            pltpu.VMEM((ROWS, D), jnp.bfloat16),
            pltpu.VMEM((ROWS, D), jnp.bfloat16),
            pltpu.VMEM((D, D), jnp.bfloat16),
            pltpu.VMEM((D, D), jnp.bfloat16),
            pltpu.VMEM((D, D), jnp.bfloat16),
            pltpu.VMEM((D, D), jnp.bfloat16),
            pltpu.VMEM((2 * NBF, BR, CW), jnp.bfloat16),
            pltpu.VMEM((2 * NBF, BR, CW), jnp.bfloat16),
            pltpu.VMEM((2 * NBF, BUF_ROWS, CW), jnp.bfloat16),
            pltpu.SemaphoreType.DMA((64,)),
            pltpu.SemaphoreType.DMA((64,)),
        ],
        compiler_params=pltpu.CompilerParams(
            collective_id=0, vmem_limit_bytes=100 * 1024 * 1024),
    )(x2, Wq, Wk, Wv, Wo)
    return out.reshape(B, SQ, D)
